# Initial kernel scaffold; baseline (speedup 1.0000x reference)
import jax
import jax.numpy as jnp
from jax import lax
from jax.experimental import pallas as pl
from jax.experimental.pallas import tpu as pltpu
from jax.experimental.pallas import tpu_sc as plsc

N = 10000
D = 128
E = 320000
P = 16
OUT = 128

NW = 32
EPW = E // NW
CHUNK = 400
NB_PREP = 8
GB = 1000
BE = 2000


def _prep_body(x_ref, wcat_ref, bcat_ref, wh1_ref, wh2_ref,
               h1_ref, h2_ref, z4_ref, xh_ref):
    xb = x_ref[...]
    z = jnp.maximum(
        jnp.dot(xb, wcat_ref[...], preferred_element_type=jnp.float32)
        + bcat_ref[...], 0.0)
    h1_ref[...] = jnp.dot(z, wh1_ref[...], preferred_element_type=jnp.float32)
    h2_ref[...] = jnp.dot(z, wh2_ref[...], preferred_element_type=jnp.float32)
    z4_ref[...] = z[:, 48:64]
    n2 = jnp.sum(xb * xb, axis=1, keepdims=True)
    xh_ref[...] = xb / jnp.maximum(jnp.sqrt(n2), 1e-8)


def _prep_call(x, wcat, bcat, wh1, wh2):
    rb = N // NB_PREP
    return pl.pallas_call(
        _prep_body,
        grid=(NB_PREP,),
        in_specs=[
            pl.BlockSpec((rb, D), lambda i: (i, 0)),
            pl.BlockSpec((D, 64), lambda i: (0, 0)),
            pl.BlockSpec((1, 64), lambda i: (0, 0)),
            pl.BlockSpec((64, OUT), lambda i: (0, 0)),
            pl.BlockSpec((64, OUT), lambda i: (0, 0)),
        ],
        out_specs=[
            pl.BlockSpec((rb, OUT), lambda i: (i, 0)),
            pl.BlockSpec((rb, OUT), lambda i: (i, 0)),
            pl.BlockSpec((rb, P), lambda i: (i, 0)),
            pl.BlockSpec((rb, D), lambda i: (i, 0)),
        ],
        out_shape=[
            jax.ShapeDtypeStruct((N, OUT), jnp.float32),
            jax.ShapeDtypeStruct((N, OUT), jnp.float32),
            jax.ShapeDtypeStruct((N, P), jnp.float32),
            jax.ShapeDtypeStruct((N, D), jnp.float32),
        ],
    )(x, wcat, bcat, wh1, wh2)


def _gram_body(xa_ref, xb_ref, out_ref):
    a = xa_ref[...].astype(jnp.bfloat16)
    b = xb_ref[...].astype(jnp.bfloat16)
    out_ref[...] = lax.dot_general(
        a, b, (((1,), (1,)), ((), ())), preferred_element_type=jnp.float32)


def _gram_call(xh):
    nb = N // GB
    return pl.pallas_call(
        _gram_body,
        grid=(nb, nb),
        in_specs=[
            pl.BlockSpec((GB, D), lambda i, j: (i, 0)),
            pl.BlockSpec((GB, D), lambda i, j: (j, 0)),
        ],
        out_specs=pl.BlockSpec((GB, GB), lambda i, j: (i, j)),
        out_shape=jax.ShapeDtypeStruct((N, N), jnp.float32),
    )(xh, xh)


def _sc_body(h1, h2, z4, gflat, rowv, colv,
             ga, gb, pa, pb, sv,
             idxr, idxc, fidx, bufa, bufb, p4a, p4b, sbuf, sem):
    cid = lax.axis_index("c")
    sid = lax.axis_index("s")
    wid = sid * 2 + cid
    base0 = wid * EPW

    def chunk_body(ci, carry):
        base = base0 + ci * CHUNK
        pltpu.sync_copy(rowv.at[pl.ds(base, CHUNK)], idxr)
        pltpu.sync_copy(colv.at[pl.ds(base, CHUNK)], idxc)
        for i in range(CHUNK // 16):
            r = idxr[pl.ds(i * 16, 16)]
            c2 = idxc[pl.ds(i * 16, 16)]
            fidx[pl.ds(i * 16, 16)] = r * N + c2
        d1 = pltpu.async_copy(h1.at[idxr], bufa, sem)
        d2 = pltpu.async_copy(h2.at[idxc], bufb, sem)
        d3 = pltpu.async_copy(z4.at[idxr], p4a, sem)
        d4 = pltpu.async_copy(z4.at[idxc], p4b, sem)
        d5 = pltpu.async_copy(gflat.at[fidx], sbuf, sem)
        d1.wait()
        d2.wait()
        d3.wait()
        d4.wait()
        d5.wait()
        pltpu.sync_copy(bufa, ga.at[pl.ds(base, CHUNK)])
        pltpu.sync_copy(bufb, gb.at[pl.ds(base, CHUNK)])
        pltpu.sync_copy(p4a, pa.at[pl.ds(base, CHUNK)])
        pltpu.sync_copy(p4b, pb.at[pl.ds(base, CHUNK)])
        pltpu.sync_copy(sbuf, sv.at[pl.ds(base, CHUNK)])
        return carry

    lax.fori_loop(0, EPW // CHUNK, chunk_body, 0)


def _sc_call(h1, h2, z4, gflat, rowv, colv):
    mesh = plsc.VectorSubcoreMesh(core_axis_name="c", subcore_axis_name="s")
    fn = pl.kernel(
        _sc_body,
        out_type=[
            jax.ShapeDtypeStruct((E, OUT), jnp.float32),
            jax.ShapeDtypeStruct((E, OUT), jnp.float32),
            jax.ShapeDtypeStruct((E, P), jnp.float32),
            jax.ShapeDtypeStruct((E, P), jnp.float32),
            jax.ShapeDtypeStruct((E, 1), jnp.float32),
        ],
        mesh=mesh,
        scratch_types=[
            pltpu.VMEM((CHUNK,), jnp.int32),
            pltpu.VMEM((CHUNK,), jnp.int32),
            pltpu.VMEM((CHUNK,), jnp.int32),
            pltpu.VMEM((CHUNK, OUT), jnp.float32),
            pltpu.VMEM((CHUNK, OUT), jnp.float32),
            pltpu.VMEM((CHUNK, P), jnp.float32),
            pltpu.VMEM((CHUNK, P), jnp.float32),
            pltpu.VMEM((CHUNK, 1), jnp.float32),
            pltpu.SemaphoreType.DMA,
        ],
    )
    return fn(h1, h2, z4, gflat, rowv, colv)


def _final_body(ga_ref, gb_ref, pa_ref, pb_ref, sv_ref, ea_ref,
                dw_ref, fw_ref, wfs_ref, bf_ref, out_ref):
    acc = ga_ref[...] + gb_ref[...]
    p = pa_ref[...] * pb_ref[...]
    acc = acc + jnp.dot(jnp.sqrt(p), dw_ref[...],
                        preferred_element_type=jnp.float32)
    acc = acc + jnp.dot(ea_ref[...], fw_ref[...],
                        preferred_element_type=jnp.float32)
    acc = acc + sv_ref[...] * wfs_ref[...]
    acc = acc + bf_ref[...]
    out_ref[...] = jnp.maximum(acc, 0.0)


def _final_call(ga, gb, pa, pb, sv, ea, dw, fw, wfs, bfv):
    nb = E // BE
    return pl.pallas_call(
        _final_body,
        grid=(nb,),
        in_specs=[
            pl.BlockSpec((BE, OUT), lambda i: (i, 0)),
            pl.BlockSpec((BE, OUT), lambda i: (i, 0)),
            pl.BlockSpec((BE, P), lambda i: (i, 0)),
            pl.BlockSpec((BE, P), lambda i: (i, 0)),
            pl.BlockSpec((BE, 1), lambda i: (i, 0)),
            pl.BlockSpec((BE, P), lambda i: (i, 0)),
            pl.BlockSpec((P, OUT), lambda i: (0, 0)),
            pl.BlockSpec((P, OUT), lambda i: (0, 0)),
            pl.BlockSpec((1, OUT), lambda i: (0, 0)),
            pl.BlockSpec((1, OUT), lambda i: (0, 0)),
        ],
        out_specs=pl.BlockSpec((BE, OUT), lambda i: (i, 0)),
        out_shape=jax.ShapeDtypeStruct((E, OUT), jnp.float32),
    )(ga, gb, pa, pb, sv, ea, dw, fw, wfs, bfv)


def kernel(x, edge_index, edge_attr, W1, b1, W2, b2, W3, b3, W4, b4, Wf, bf):
    row = edge_index[0].astype(jnp.int32)
    col = edge_index[1].astype(jnp.int32)
    wcat = jnp.concatenate([W1, W2, W3, W4], axis=1)
    bcat = jnp.concatenate([b1, b2, b3, b4]).reshape(1, 4 * P)
    A = Wf[0:P]
    Bw = Wf[P:2 * P]
    Cw = Wf[2 * P:3 * P]
    Dw = Wf[3 * P:4 * P]
    wfs = Wf[4 * P:4 * P + 1]
    Fw = Wf[4 * P + 1:]
    wh1 = jnp.zeros((4 * P, OUT), jnp.float32).at[0:P].set(A).at[2 * P:3 * P].set(Cw)
    wh2 = jnp.zeros((4 * P, OUT), jnp.float32).at[P:2 * P].set(Bw).at[2 * P:3 * P].set(-Cw)
    h1, h2, z4, xh = _prep_call(x, wcat, bcat, wh1, wh2)
    ghat = _gram_call(xh)
    gflat = ghat.reshape(N * N, 1)
    ga, gb, pa, pb, sv = _sc_call(h1, h2, z4, gflat, row, col)
    out = _final_call(ga, gb, pa, pb, sv, edge_attr, Dw, Fw, wfs,
                      bf.reshape(1, OUT))
    return out


# speedup vs baseline: 3.9012x; 3.9012x over previous
import jax
import jax.numpy as jnp
from jax import lax
from jax.experimental import pallas as pl
from jax.experimental.pallas import tpu as pltpu
from jax.experimental.pallas import tpu_sc as plsc

N = 10000
D = 128
E = 320000
P = 16
OUT = 128

NW = 32
EPW = E // NW
CHUNK = 80
NB_PREP = 10
GB = 80
BE = 2000


def _prep_body(x_ref, wcat_ref, bcat_ref, wh1_ref, wh2_ref,
               h1_ref, h2_ref, z4_ref, xh_ref):
    xb = x_ref[...]
    z = jnp.maximum(
        jnp.dot(xb, wcat_ref[...], preferred_element_type=jnp.float32)
        + bcat_ref[...], 0.0)
    h1_ref[...] = jnp.dot(z, wh1_ref[...], preferred_element_type=jnp.float32)
    h2_ref[...] = jnp.dot(z, wh2_ref[...], preferred_element_type=jnp.float32)
    rb = z.shape[0]
    z4_ref[...] = jnp.concatenate(
        [z[:, 48:64], jnp.zeros((rb, 112), jnp.float32)], axis=1)
    n2 = jnp.sum(xb * xb, axis=1, keepdims=True)
    xh_ref[...] = xb / jnp.maximum(jnp.sqrt(n2), 1e-8)


def _prep_call(x, wcat, bcat, wh1, wh2):
    rb = N // NB_PREP
    return pl.pallas_call(
        _prep_body,
        grid=(NB_PREP,),
        in_specs=[
            pl.BlockSpec((rb, D), lambda i: (i, 0)),
            pl.BlockSpec((D, 64), lambda i: (0, 0)),
            pl.BlockSpec((1, 64), lambda i: (0, 0)),
            pl.BlockSpec((64, OUT), lambda i: (0, 0)),
            pl.BlockSpec((64, OUT), lambda i: (0, 0)),
        ],
        out_specs=[
            pl.BlockSpec((rb, OUT), lambda i: (i, 0)),
            pl.BlockSpec((rb, OUT), lambda i: (i, 0)),
            pl.BlockSpec((rb, D), lambda i: (i, 0)),
            pl.BlockSpec((rb, D), lambda i: (i, 0)),
        ],
        out_shape=[
            jax.ShapeDtypeStruct((N, OUT), jnp.float32),
            jax.ShapeDtypeStruct((N, OUT), jnp.float32),
            jax.ShapeDtypeStruct((N, D), jnp.float32),
            jax.ShapeDtypeStruct((N, D), jnp.float32),
        ],
    )(x, wcat, bcat, wh1, wh2)


def _gram_body(xa_ref, xb_ref, out_ref):
    a = xa_ref[...].astype(jnp.bfloat16)
    b = xb_ref[...].astype(jnp.bfloat16)
    out_ref[...] = lax.dot_general(
        a, b, (((1,), (1,)), ((), ())), preferred_element_type=jnp.float32)


def _gram_call(xh):
    nb = N // GB
    return pl.pallas_call(
        _gram_body,
        grid=(nb,),
        in_specs=[
            pl.BlockSpec((GB, D), lambda i: (i, 0)),
            pl.BlockSpec((N, D), lambda i: (0, 0)),
        ],
        out_specs=pl.BlockSpec((GB, N), lambda i: (i, 0)),
        out_shape=jax.ShapeDtypeStruct((N, N), jnp.float32),
    )(xh, xh)


def _sc_body(h1, h2, z4, gflat, rowv, colv,
             ga, gb, pp, sv,
             idxr, idxc, fidx, bufa, bufb, p4a, p4b, prod, sbuf, sem):
    cid = lax.axis_index("c")
    sid = lax.axis_index("s")
    wid = sid * 2 + cid
    base0 = wid * EPW

    def chunk_body(ci, carry):
        base = base0 + ci * CHUNK
        pltpu.sync_copy(rowv.at[pl.ds(base, CHUNK)], idxr)
        pltpu.sync_copy(colv.at[pl.ds(base, CHUNK)], idxc)
        for i in range(CHUNK // 16):
            r = idxr[pl.ds(i * 16, 16)]
            c2 = idxc[pl.ds(i * 16, 16)]
            fidx[pl.ds(i * 16, 16)] = r * N + c2
        d1 = pltpu.async_copy(h1.at[idxr], bufa, sem)
        d2 = pltpu.async_copy(h2.at[idxc], bufb, sem)
        d3 = pltpu.async_copy(z4.at[idxr], p4a, sem)
        d4 = pltpu.async_copy(z4.at[idxc], p4b, sem)
        d5 = pltpu.async_copy(gflat.at[fidx], sbuf, sem)
        d1.wait()
        d2.wait()
        d3.wait()
        d4.wait()
        d5.wait()
        for j in range(CHUNK):
            prod[j, :] = p4a[j, pl.ds(0, P)] * p4b[j, pl.ds(0, P)]
        pltpu.sync_copy(bufa, ga.at[pl.ds(base, CHUNK)])
        pltpu.sync_copy(bufb, gb.at[pl.ds(base, CHUNK)])
        pltpu.sync_copy(prod, pp.at[pl.ds(base, CHUNK)])
        pltpu.sync_copy(sbuf, sv.at[pl.ds(base, CHUNK)])
        return carry

    lax.fori_loop(0, EPW // CHUNK, chunk_body, 0)


def _sc_call(h1, h2, z4, gflat, rowv, colv):
    mesh = plsc.VectorSubcoreMesh(core_axis_name="c", subcore_axis_name="s")
    fn = pl.kernel(
        _sc_body,
        out_type=[
            jax.ShapeDtypeStruct((E, OUT), jnp.float32),
            jax.ShapeDtypeStruct((E, OUT), jnp.float32),
            jax.ShapeDtypeStruct((E, P), jnp.float32),
            jax.ShapeDtypeStruct((E,), jnp.float32),
        ],
        mesh=mesh,
        scratch_types=[
            pltpu.VMEM((CHUNK,), jnp.int32),
            pltpu.VMEM((CHUNK,), jnp.int32),
            pltpu.VMEM((CHUNK,), jnp.int32),
            pltpu.VMEM((CHUNK, OUT), jnp.float32),
            pltpu.VMEM((CHUNK, OUT), jnp.float32),
            pltpu.VMEM((CHUNK, D), jnp.float32),
            pltpu.VMEM((CHUNK, D), jnp.float32),
            pltpu.VMEM((CHUNK, P), jnp.float32),
            pltpu.VMEM((CHUNK,), jnp.float32),
            pltpu.SemaphoreType.DMA,
        ],
    )
    return fn(h1, h2, z4, gflat, rowv, colv)


def _final_body(ga_ref, gb_ref, pp_ref, sv_ref, ea_ref,
                dw_ref, fw_ref, wfs_ref, bf_ref, out_ref):
    acc = ga_ref[...] + gb_ref[...]
    p = pp_ref[...]
    acc = acc + jnp.dot(jnp.sqrt(p), dw_ref[...],
                        preferred_element_type=jnp.float32)
    acc = acc + jnp.dot(ea_ref[...], fw_ref[...],
                        preferred_element_type=jnp.float32)
    acc = acc + sv_ref[...] * wfs_ref[...]
    acc = acc + bf_ref[...]
    out_ref[...] = jnp.maximum(acc, 0.0)


def _final_call(ga, gb, pp, sv, ea, dw, fw, wfs, bfv):
    nb = E // BE
    return pl.pallas_call(
        _final_body,
        grid=(nb,),
        in_specs=[
            pl.BlockSpec((BE, OUT), lambda i: (i, 0)),
            pl.BlockSpec((BE, OUT), lambda i: (i, 0)),
            pl.BlockSpec((BE, P), lambda i: (i, 0)),
            pl.BlockSpec((BE, 1), lambda i: (i, 0)),
            pl.BlockSpec((BE, P), lambda i: (i, 0)),
            pl.BlockSpec((P, OUT), lambda i: (0, 0)),
            pl.BlockSpec((P, OUT), lambda i: (0, 0)),
            pl.BlockSpec((1, OUT), lambda i: (0, 0)),
            pl.BlockSpec((1, OUT), lambda i: (0, 0)),
        ],
        out_specs=pl.BlockSpec((BE, OUT), lambda i: (i, 0)),
        out_shape=jax.ShapeDtypeStruct((E, OUT), jnp.float32),
    )(ga, gb, pp, sv, ea, dw, fw, wfs, bfv)


def kernel(x, edge_index, edge_attr, W1, b1, W2, b2, W3, b3, W4, b4, Wf, bf):
    row = edge_index[0].astype(jnp.int32)
    col = edge_index[1].astype(jnp.int32)
    wcat = jnp.concatenate([W1, W2, W3, W4], axis=1)
    bcat = jnp.concatenate([b1, b2, b3, b4]).reshape(1, 4 * P)
    A = Wf[0:P]
    Bw = Wf[P:2 * P]
    Cw = Wf[2 * P:3 * P]
    Dw = Wf[3 * P:4 * P]
    wfs = Wf[4 * P:4 * P + 1]
    Fw = Wf[4 * P + 1:]
    wh1 = jnp.zeros((4 * P, OUT), jnp.float32).at[0:P].set(A).at[2 * P:3 * P].set(Cw)
    wh2 = jnp.zeros((4 * P, OUT), jnp.float32).at[P:2 * P].set(Bw).at[2 * P:3 * P].set(-Cw)
    h1, h2, z4, xh = _prep_call(x, wcat, bcat, wh1, wh2)
    ghat = _gram_call(xh)
    gflat = ghat.reshape(N * N)
    ga, gb, pp, sv = _sc_call(h1, h2, z4, gflat, row, col)
    out = _final_call(ga, gb, pp, sv.reshape(E, 1), edge_attr, Dw, Fw, wfs,
                      bf.reshape(1, OUT))
    return out


# speedup vs baseline: 4.7551x; 1.2189x over previous
import jax
import jax.numpy as jnp
from jax import lax
from jax.experimental import pallas as pl
from jax.experimental.pallas import tpu as pltpu
from jax.experimental.pallas import tpu_sc as plsc

N = 10000
D = 128
E = 320000
P = 16
OUT = 128

NW = 32
EPW = E // NW
CHUNK = 400
NB_PREP = 10
GB = 80
BE = 2000


def _prep_body(x_ref, wcat_ref, bcat_ref, wh1_ref, wh2_ref,
               t1_ref, t2_ref, xh_ref):
    xb = x_ref[...]
    z = jnp.maximum(
        jnp.dot(xb, wcat_ref[...], preferred_element_type=jnp.float32)
        + bcat_ref[...], 0.0)
    h1 = jnp.dot(z, wh1_ref[...], preferred_element_type=jnp.float32)
    h2 = jnp.dot(z, wh2_ref[...], preferred_element_type=jnp.float32)
    z4 = z[:, 48:64]
    rb = z.shape[0]

    def pk(a):
        half = a.shape[1] // 2
        lo = lax.bitcast_convert_type(
            a[:, :half].astype(jnp.bfloat16), jnp.uint16).astype(jnp.uint32)
        hi = lax.bitcast_convert_type(
            a[:, half:].astype(jnp.bfloat16), jnp.uint16).astype(jnp.uint32)
        return lax.bitcast_convert_type(lo | (hi << 16), jnp.int32)

    pad = jnp.zeros((rb, 56), jnp.int32)
    t1_ref[...] = jnp.concatenate([pk(h1), pk(z4), pad], axis=1)
    t2_ref[...] = jnp.concatenate([pk(h2), pk(z4), pad], axis=1)
    n2 = jnp.sum(xb * xb, axis=1, keepdims=True)
    xh_ref[...] = xb / jnp.maximum(jnp.sqrt(n2), 1e-8)


def _prep_call(x, wcat, bcat, wh1, wh2):
    rb = N // NB_PREP
    return pl.pallas_call(
        _prep_body,
        grid=(NB_PREP,),
        in_specs=[
            pl.BlockSpec((rb, D), lambda i: (i, 0)),
            pl.BlockSpec((D, 64), lambda i: (0, 0)),
            pl.BlockSpec((1, 64), lambda i: (0, 0)),
            pl.BlockSpec((64, OUT), lambda i: (0, 0)),
            pl.BlockSpec((64, OUT), lambda i: (0, 0)),
        ],
        out_specs=[
            pl.BlockSpec((rb, D), lambda i: (i, 0)),
            pl.BlockSpec((rb, D), lambda i: (i, 0)),
            pl.BlockSpec((rb, D), lambda i: (i, 0)),
        ],
        out_shape=[
            jax.ShapeDtypeStruct((N, D), jnp.int32),
            jax.ShapeDtypeStruct((N, D), jnp.int32),
            jax.ShapeDtypeStruct((N, D), jnp.float32),
        ],
    )(x, wcat, bcat, wh1, wh2)


def _gram_body(xa_ref, xb_ref, out_ref):
    a = xa_ref[...].astype(jnp.bfloat16)
    b = xb_ref[...].astype(jnp.bfloat16)
    out_ref[...] = lax.dot_general(
        a, b, (((1,), (1,)), ((), ())), preferred_element_type=jnp.float32)


def _gram_call(xh):
    nb = N // GB
    return pl.pallas_call(
        _gram_body,
        grid=(nb,),
        in_specs=[
            pl.BlockSpec((GB, D), lambda i: (i, 0)),
            pl.BlockSpec((N, D), lambda i: (0, 0)),
        ],
        out_specs=pl.BlockSpec((GB, N), lambda i: (i, 0)),
        out_shape=jax.ShapeDtypeStruct((N, N), jnp.float32),
    )(xh, xh)


def _sc_body(t1, t2, gflat, rowv, colv,
             ga, gb, sv,
             idxr, idxc, fidx, bufa, bufb, sbuf, sem):
    cid = lax.axis_index("c")
    sid = lax.axis_index("s")
    wid = sid * 2 + cid
    base0 = wid * EPW

    def chunk_body(ci, carry):
        base = base0 + ci * CHUNK
        pltpu.sync_copy(rowv.at[pl.ds(base, CHUNK)], idxr)
        pltpu.sync_copy(colv.at[pl.ds(base, CHUNK)], idxc)
        for i in range(CHUNK // 16):
            r = idxr[pl.ds(i * 16, 16)]
            c2 = idxc[pl.ds(i * 16, 16)]
            fidx[pl.ds(i * 16, 16)] = r * N + c2
        d1 = pltpu.async_copy(t1.at[idxr], bufa, sem)
        d2 = pltpu.async_copy(t2.at[idxc], bufb, sem)
        d5 = pltpu.async_copy(gflat.at[fidx], sbuf, sem)
        d1.wait()
        d2.wait()
        d5.wait()
        pltpu.sync_copy(bufa, ga.at[pl.ds(base, CHUNK)])
        pltpu.sync_copy(bufb, gb.at[pl.ds(base, CHUNK)])
        pltpu.sync_copy(sbuf, sv.at[pl.ds(base, CHUNK)])
        return carry

    lax.fori_loop(0, EPW // CHUNK, chunk_body, 0)


def _sc_call(t1, t2, gflat, rowv, colv):
    mesh = plsc.VectorSubcoreMesh(core_axis_name="c", subcore_axis_name="s")
    fn = pl.kernel(
        _sc_body,
        out_type=[
            jax.ShapeDtypeStruct((E, D), jnp.int32),
            jax.ShapeDtypeStruct((E, D), jnp.int32),
            jax.ShapeDtypeStruct((E,), jnp.float32),
        ],
        mesh=mesh,
        scratch_types=[
            pltpu.VMEM((CHUNK,), jnp.int32),
            pltpu.VMEM((CHUNK,), jnp.int32),
            pltpu.VMEM((CHUNK,), jnp.int32),
            pltpu.VMEM((CHUNK, D), jnp.int32),
            pltpu.VMEM((CHUNK, D), jnp.int32),
            pltpu.VMEM((CHUNK,), jnp.float32),
            pltpu.SemaphoreType.DMA,
        ],
    )
    return fn(t1, t2, gflat, rowv, colv)


def _unpk(w):
    lo = lax.bitcast_convert_type(lax.shift_left(w, 16), jnp.float32)
    hi = lax.bitcast_convert_type(w & jnp.int32(-65536), jnp.float32)
    return jnp.concatenate([lo, hi], axis=1)


def _final_body(ga_ref, gb_ref, sv_ref, ea_ref,
                dw_ref, fw_ref, wfs_ref, bf_ref, out_ref):
    ga = ga_ref[...]
    gb = gb_ref[...]
    h1r = _unpk(ga[:, 0:64])
    h2c = _unpk(gb[:, 0:64])
    z4r = _unpk(ga[:, 64:72])
    z4c = _unpk(gb[:, 64:72])
    acc = h1r + h2c
    p = z4r * z4c
    acc = acc + jnp.dot(jnp.sqrt(p), dw_ref[...],
                        preferred_element_type=jnp.float32)
    acc = acc + jnp.dot(ea_ref[...], fw_ref[...],
                        preferred_element_type=jnp.float32)
    acc = acc + sv_ref[...] * wfs_ref[...]
    acc = acc + bf_ref[...]
    out_ref[...] = jnp.maximum(acc, 0.0)


def _final_call(ga, gb, sv, ea, dw, fw, wfs, bfv):
    nb = E // BE
    return pl.pallas_call(
        _final_body,
        grid=(nb,),
        in_specs=[
            pl.BlockSpec((BE, D), lambda i: (i, 0)),
            pl.BlockSpec((BE, D), lambda i: (i, 0)),
            pl.BlockSpec((BE, 1), lambda i: (i, 0)),
            pl.BlockSpec((BE, P), lambda i: (i, 0)),
            pl.BlockSpec((P, OUT), lambda i: (0, 0)),
            pl.BlockSpec((P, OUT), lambda i: (0, 0)),
            pl.BlockSpec((1, OUT), lambda i: (0, 0)),
            pl.BlockSpec((1, OUT), lambda i: (0, 0)),
        ],
        out_specs=pl.BlockSpec((BE, OUT), lambda i: (i, 0)),
        out_shape=jax.ShapeDtypeStruct((E, OUT), jnp.float32),
    )(ga, gb, sv, ea, dw, fw, wfs, bfv)


def kernel(x, edge_index, edge_attr, W1, b1, W2, b2, W3, b3, W4, b4, Wf, bf):
    row = edge_index[0].astype(jnp.int32)
    col = edge_index[1].astype(jnp.int32)
    wcat = jnp.concatenate([W1, W2, W3, W4], axis=1)
    bcat = jnp.concatenate([b1, b2, b3, b4]).reshape(1, 4 * P)
    A = Wf[0:P]
    Bw = Wf[P:2 * P]
    Cw = Wf[2 * P:3 * P]
    Dw = Wf[3 * P:4 * P]
    wfs = Wf[4 * P:4 * P + 1]
    Fw = Wf[4 * P + 1:]
    wh1 = jnp.zeros((4 * P, OUT), jnp.float32).at[0:P].set(A).at[2 * P:3 * P].set(Cw)
    wh2 = jnp.zeros((4 * P, OUT), jnp.float32).at[P:2 * P].set(Bw).at[2 * P:3 * P].set(-Cw)
    t1, t2, xh = _prep_call(x, wcat, bcat, wh1, wh2)
    ghat = _gram_call(xh)
    gflat = ghat.reshape(N * N)
    ga, gb, sv = _sc_call(t1, t2, gflat, row, col)
    out = _final_call(ga, gb, sv.reshape(E, 1), edge_attr, Dw, Fw, wfs,
                      bf.reshape(1, OUT))
    return out


# speedup vs baseline: 6.5955x; 1.3870x over previous
import jax
import jax.numpy as jnp
from jax import lax
from jax.experimental import pallas as pl
from jax.experimental.pallas import tpu as pltpu
from jax.experimental.pallas import tpu_sc as plsc

N = 10000
D = 128
E = 320000
P = 16
OUT = 128

NW = 32
EPW = E // NW
CHUNK = 400
NB_PREP = 10
GB = 80
BE = 2000


def _prep_body(x_ref, wcat_ref, bcat_ref, wh1_ref, wh2_ref,
               t1_ref, t2_ref):
    xb = x_ref[...]
    z = jnp.maximum(
        jnp.dot(xb, wcat_ref[...], preferred_element_type=jnp.float32)
        + bcat_ref[...], 0.0)
    h1 = jnp.dot(z, wh1_ref[...], preferred_element_type=jnp.float32)
    h2 = jnp.dot(z, wh2_ref[...], preferred_element_type=jnp.float32)
    z4 = z[:, 48:64]
    rb = z.shape[0]

    def pk(a):
        half = a.shape[1] // 2
        lo = lax.bitcast_convert_type(
            a[:, :half].astype(jnp.bfloat16), jnp.uint16).astype(jnp.uint32)
        hi = lax.bitcast_convert_type(
            a[:, half:].astype(jnp.bfloat16), jnp.uint16).astype(jnp.uint32)
        return lax.bitcast_convert_type(lo | (hi << 16), jnp.int32)

    n2 = jnp.sum(xb * xb, axis=1, keepdims=True)
    xh = xb / jnp.maximum(jnp.sqrt(n2), 1e-8)
    xq = lax.bitcast_convert_type(
        xh.astype(jnp.float8_e4m3fn), jnp.uint8).astype(jnp.uint32)
    xw = lax.bitcast_convert_type(
        xq[:, 0:32] | (xq[:, 32:64] << 8) | (xq[:, 64:96] << 16)
        | (xq[:, 96:128] << 24), jnp.int32)
    pad = jnp.zeros((rb, 24), jnp.int32)
    t1_ref[...] = jnp.concatenate([pk(h1), pk(z4), xw, pad], axis=1)
    t2_ref[...] = jnp.concatenate([pk(h2), pk(z4), xw, pad], axis=1)


def _prep_call(x, wcat, bcat, wh1, wh2):
    rb = N // NB_PREP
    return pl.pallas_call(
        _prep_body,
        grid=(NB_PREP,),
        in_specs=[
            pl.BlockSpec((rb, D), lambda i: (i, 0)),
            pl.BlockSpec((D, 64), lambda i: (0, 0)),
            pl.BlockSpec((1, 64), lambda i: (0, 0)),
            pl.BlockSpec((64, OUT), lambda i: (0, 0)),
            pl.BlockSpec((64, OUT), lambda i: (0, 0)),
        ],
        out_specs=[
            pl.BlockSpec((rb, D), lambda i: (i, 0)),
            pl.BlockSpec((rb, D), lambda i: (i, 0)),
        ],
        out_shape=[
            jax.ShapeDtypeStruct((N, D), jnp.int32),
            jax.ShapeDtypeStruct((N, D), jnp.int32),
        ],
    )(x, wcat, bcat, wh1, wh2)


def _sc_body(t1, t2, rowv, colv,
             ga, gb,
             idxr, idxc, bufa, bufb, sem):
    cid = lax.axis_index("c")
    sid = lax.axis_index("s")
    wid = sid * 2 + cid
    base0 = wid * EPW

    def chunk_body(ci, carry):
        base = base0 + ci * CHUNK
        pltpu.sync_copy(rowv.at[pl.ds(base, CHUNK)], idxr)
        pltpu.sync_copy(colv.at[pl.ds(base, CHUNK)], idxc)
        d1 = pltpu.async_copy(t1.at[idxr], bufa, sem)
        d2 = pltpu.async_copy(t2.at[idxc], bufb, sem)
        d1.wait()
        d2.wait()
        pltpu.sync_copy(bufa, ga.at[pl.ds(base, CHUNK)])
        pltpu.sync_copy(bufb, gb.at[pl.ds(base, CHUNK)])
        return carry

    lax.fori_loop(0, EPW // CHUNK, chunk_body, 0)


def _sc_call(t1, t2, rowv, colv):
    mesh = plsc.VectorSubcoreMesh(core_axis_name="c", subcore_axis_name="s")
    fn = pl.kernel(
        _sc_body,
        out_type=[
            jax.ShapeDtypeStruct((E, D), jnp.int32),
            jax.ShapeDtypeStruct((E, D), jnp.int32),
        ],
        mesh=mesh,
        scratch_types=[
            pltpu.VMEM((CHUNK,), jnp.int32),
            pltpu.VMEM((CHUNK,), jnp.int32),
            pltpu.VMEM((CHUNK, D), jnp.int32),
            pltpu.VMEM((CHUNK, D), jnp.int32),
            pltpu.SemaphoreType.DMA,
        ],
    )
    return fn(t1, t2, rowv, colv)


def _unpk(w):
    lo = lax.bitcast_convert_type(lax.shift_left(w, 16), jnp.float32)
    hi = lax.bitcast_convert_type(w & jnp.int32(-65536), jnp.float32)
    return jnp.concatenate([lo, hi], axis=1)


def _unpk8(w):
    parts = []
    for k in range(4):
        b = lax.shift_right_logical(w, 8 * k) & jnp.int32(0xFF)
        f8 = lax.bitcast_convert_type(b.astype(jnp.uint8), jnp.float8_e4m3fn)
        parts.append(f8.astype(jnp.float32))
    return jnp.concatenate(parts, axis=1)


def _final_body(ga_ref, gb_ref, ea_ref,
                dw_ref, fw_ref, wfs_ref, bf_ref, out_ref):
    ga = ga_ref[...]
    gb = gb_ref[...]
    h1r = _unpk(ga[:, 0:64])
    h2c = _unpk(gb[:, 0:64])
    z4r = _unpk(ga[:, 64:72])
    z4c = _unpk(gb[:, 64:72])
    xr = _unpk8(ga[:, 72:104])
    xc = _unpk8(gb[:, 72:104])
    s = jnp.sum(xr * xc, axis=1, keepdims=True)
    acc = h1r + h2c
    p = z4r * z4c
    acc = acc + jnp.dot(jnp.sqrt(p), dw_ref[...],
                        preferred_element_type=jnp.float32)
    acc = acc + jnp.dot(ea_ref[...], fw_ref[...],
                        preferred_element_type=jnp.float32)
    acc = acc + s * wfs_ref[...]
    acc = acc + bf_ref[...]
    out_ref[...] = jnp.maximum(acc, 0.0)


def _final_call(ga, gb, ea, dw, fw, wfs, bfv):
    nb = E // BE
    return pl.pallas_call(
        _final_body,
        grid=(nb,),
        in_specs=[
            pl.BlockSpec((BE, D), lambda i: (i, 0)),
            pl.BlockSpec((BE, D), lambda i: (i, 0)),
            pl.BlockSpec((BE, P), lambda i: (i, 0)),
            pl.BlockSpec((P, OUT), lambda i: (0, 0)),
            pl.BlockSpec((P, OUT), lambda i: (0, 0)),
            pl.BlockSpec((1, OUT), lambda i: (0, 0)),
            pl.BlockSpec((1, OUT), lambda i: (0, 0)),
        ],
        out_specs=pl.BlockSpec((BE, OUT), lambda i: (i, 0)),
        out_shape=jax.ShapeDtypeStruct((E, OUT), jnp.float32),
    )(ga, gb, ea, dw, fw, wfs, bfv)


def kernel(x, edge_index, edge_attr, W1, b1, W2, b2, W3, b3, W4, b4, Wf, bf):
    row = edge_index[0].astype(jnp.int32)
    col = edge_index[1].astype(jnp.int32)
    wcat = jnp.concatenate([W1, W2, W3, W4], axis=1)
    bcat = jnp.concatenate([b1, b2, b3, b4]).reshape(1, 4 * P)
    A = Wf[0:P]
    Bw = Wf[P:2 * P]
    Cw = Wf[2 * P:3 * P]
    Dw = Wf[3 * P:4 * P]
    wfs = Wf[4 * P:4 * P + 1]
    Fw = Wf[4 * P + 1:]
    wh1 = jnp.zeros((4 * P, OUT), jnp.float32).at[0:P].set(A).at[2 * P:3 * P].set(Cw)
    wh2 = jnp.zeros((4 * P, OUT), jnp.float32).at[P:2 * P].set(Bw).at[2 * P:3 * P].set(-Cw)
    t1, t2 = _prep_call(x, wcat, bcat, wh1, wh2)
    ga, gb = _sc_call(t1, t2, row, col)
    out = _final_call(ga, gb, edge_attr, Dw, Fw, wfs, bf.reshape(1, OUT))
    return out


# speedup vs baseline: 8.4686x; 1.2840x over previous
import jax
import jax.numpy as jnp
from jax import lax
from jax.experimental import pallas as pl
from jax.experimental.pallas import tpu as pltpu
from jax.experimental.pallas import tpu_sc as plsc

N = 10000
D = 128
E = 320000
P = 16
OUT = 128

NW = 32
EPW = E // NW
CHUNK = 400
NB_PREP = 10
GB = 80
BE = 2000


def _prep_body(x_ref, wcat_ref, bcat_ref, wh1_ref, wh2_ref,
               t1_ref, t2_ref):
    xb = x_ref[...]
    z = jnp.maximum(
        jnp.dot(xb, wcat_ref[...], preferred_element_type=jnp.float32)
        + bcat_ref[...], 0.0)
    h1 = jnp.dot(z, wh1_ref[...], preferred_element_type=jnp.float32)
    h2 = jnp.dot(z, wh2_ref[...], preferred_element_type=jnp.float32)
    z4 = z[:, 48:64]
    rb = z.shape[0]

    def pk(a):
        half = a.shape[1] // 2
        lo = lax.bitcast_convert_type(
            a[:, :half].astype(jnp.bfloat16), jnp.uint16).astype(jnp.uint32)
        hi = lax.bitcast_convert_type(
            a[:, half:].astype(jnp.bfloat16), jnp.uint16).astype(jnp.uint32)
        return lax.bitcast_convert_type(lo | (hi << 16), jnp.int32)

    n2 = jnp.sum(xb * xb, axis=1, keepdims=True)
    xh = xb / jnp.maximum(jnp.sqrt(n2), 1e-8)
    xq = lax.bitcast_convert_type(
        xh.astype(jnp.float8_e4m3fn), jnp.uint8).astype(jnp.uint32)
    xw = lax.bitcast_convert_type(
        xq[:, 0:32] | (xq[:, 32:64] << 8) | (xq[:, 64:96] << 16)
        | (xq[:, 96:128] << 24), jnp.int32)
    pad = jnp.zeros((rb, 24), jnp.int32)
    t1_ref[...] = jnp.concatenate([pk(h1), pk(z4), xw, pad], axis=1)
    t2_ref[...] = jnp.concatenate([pk(h2), pk(z4), xw, pad], axis=1)


def _prep_call(x, wcat, bcat, wh1, wh2):
    rb = N // NB_PREP
    return pl.pallas_call(
        _prep_body,
        grid=(NB_PREP,),
        in_specs=[
            pl.BlockSpec((rb, D), lambda i: (i, 0)),
            pl.BlockSpec((D, 64), lambda i: (0, 0)),
            pl.BlockSpec((1, 64), lambda i: (0, 0)),
            pl.BlockSpec((64, OUT), lambda i: (0, 0)),
            pl.BlockSpec((64, OUT), lambda i: (0, 0)),
        ],
        out_specs=[
            pl.BlockSpec((rb, D), lambda i: (i, 0)),
            pl.BlockSpec((rb, D), lambda i: (i, 0)),
        ],
        out_shape=[
            jax.ShapeDtypeStruct((N, D), jnp.int32),
            jax.ShapeDtypeStruct((N, D), jnp.int32),
        ],
    )(x, wcat, bcat, wh1, wh2)


def _sc_body(t1, t2, rowv, colv,
             ga, gb,
             idxr, idxc, bufa, bufb, sem):
    cid = lax.axis_index("c")
    sid = lax.axis_index("s")
    wid = sid * 2 + cid
    base0 = wid * EPW

    def chunk_body(ci, carry):
        base = base0 + ci * CHUNK
        pltpu.sync_copy(rowv.at[pl.ds(base, CHUNK)], idxr)
        pltpu.sync_copy(colv.at[pl.ds(base, CHUNK)], idxc)
        d1 = pltpu.async_copy(t1.at[idxr], bufa, sem)
        d2 = pltpu.async_copy(t2.at[idxc], bufb, sem)
        d1.wait()
        d2.wait()
        pltpu.sync_copy(bufa, ga.at[pl.ds(base, CHUNK)])
        pltpu.sync_copy(bufb, gb.at[pl.ds(base, CHUNK)])
        return carry

    lax.fori_loop(0, EPW // CHUNK, chunk_body, 0)


def _sc_call(t1, t2, rowv, colv):
    mesh = plsc.VectorSubcoreMesh(core_axis_name="c", subcore_axis_name="s")
    fn = pl.kernel(
        _sc_body,
        out_type=[
            jax.ShapeDtypeStruct((E, D), jnp.int32),
            jax.ShapeDtypeStruct((E, D), jnp.int32),
        ],
        mesh=mesh,
        scratch_types=[
            pltpu.VMEM((CHUNK,), jnp.int32),
            pltpu.VMEM((CHUNK,), jnp.int32),
            pltpu.VMEM((CHUNK, D), jnp.int32),
            pltpu.VMEM((CHUNK, D), jnp.int32),
            pltpu.SemaphoreType.DMA,
        ],
    )
    return fn(t1, t2, rowv, colv)


def _lo(w):
    return lax.bitcast_convert_type(lax.shift_left(w, 16), jnp.float32)


def _hi(w):
    return lax.bitcast_convert_type(w & jnp.int32(-65536), jnp.float32)


def _f8(w, k):
    b = lax.shift_right_logical(w, 8 * k) & jnp.int32(0xFF)
    return lax.bitcast_convert_type(
        b.astype(jnp.uint8), jnp.float8_e4m3fn).astype(jnp.float32)


def _final_body(ga_ref, gb_ref, ea_ref,
                dw_ref, fw_ref, wfs_ref, bf_ref, out_ref):
    ga = ga_ref[...]
    gb = gb_ref[...]
    dw = dw_ref[...]
    fw = fw_ref[...]
    wfs = wfs_ref[...]
    bf = bf_ref[...]
    ea = ea_ref[...]
    gaz = ga[:, 64:72]
    gbz = gb[:, 64:72]
    q_lo = jnp.sqrt(_lo(gaz) * _lo(gbz))
    q_hi = jnp.sqrt(_hi(gaz) * _hi(gbz))
    gax = ga[:, 72:104]
    gbx = gb[:, 72:104]
    prods = [_f8(gax, k) * _f8(gbx, k) for k in range(4)]
    for half, sl in ((0, slice(0, 64)), (1, slice(64, 128))):
        ext = _lo if half == 0 else _hi
        acc = ext(ga[:, 0:64]) + ext(gb[:, 0:64]) + bf[:, sl]
        acc = acc + jnp.dot(q_lo, dw[0:8, sl],
                            preferred_element_type=jnp.float32)
        acc = acc + jnp.dot(q_hi, dw[8:16, sl],
                            preferred_element_type=jnp.float32)
        acc = acc + jnp.dot(ea, fw[:, sl],
                            preferred_element_type=jnp.float32)
        wcos = jnp.broadcast_to(wfs[:, sl], (32, 64))
        for k in range(4):
            acc = acc + jnp.dot(prods[k], wcos,
                                preferred_element_type=jnp.float32)
        out_ref[:, sl] = jnp.maximum(acc, 0.0)


def _final_call(ga, gb, ea, dw, fw, wfs, bfv):
    nb = E // BE
    return pl.pallas_call(
        _final_body,
        grid=(nb,),
        in_specs=[
            pl.BlockSpec((BE, D), lambda i: (i, 0)),
            pl.BlockSpec((BE, D), lambda i: (i, 0)),
            pl.BlockSpec((BE, P), lambda i: (i, 0)),
            pl.BlockSpec((P, OUT), lambda i: (0, 0)),
            pl.BlockSpec((P, OUT), lambda i: (0, 0)),
            pl.BlockSpec((1, OUT), lambda i: (0, 0)),
            pl.BlockSpec((1, OUT), lambda i: (0, 0)),
        ],
        out_specs=pl.BlockSpec((BE, OUT), lambda i: (i, 0)),
        out_shape=jax.ShapeDtypeStruct((E, OUT), jnp.float32),
    )(ga, gb, ea, dw, fw, wfs, bfv)


def kernel(x, edge_index, edge_attr, W1, b1, W2, b2, W3, b3, W4, b4, Wf, bf):
    row = edge_index[0].astype(jnp.int32)
    col = edge_index[1].astype(jnp.int32)
    wcat = jnp.concatenate([W1, W2, W3, W4], axis=1)
    bcat = jnp.concatenate([b1, b2, b3, b4]).reshape(1, 4 * P)
    A = Wf[0:P]
    Bw = Wf[P:2 * P]
    Cw = Wf[2 * P:3 * P]
    Dw = Wf[3 * P:4 * P]
    wfs = Wf[4 * P:4 * P + 1]
    Fw = Wf[4 * P + 1:]
    wh1 = jnp.zeros((4 * P, OUT), jnp.float32).at[0:P].set(A).at[2 * P:3 * P].set(Cw)
    wh2 = jnp.zeros((4 * P, OUT), jnp.float32).at[P:2 * P].set(Bw).at[2 * P:3 * P].set(-Cw)
    t1, t2 = _prep_call(x, wcat, bcat, wh1, wh2)
    ga, gb = _sc_call(t1, t2, row, col)
    out = _final_call(ga, gb, edge_attr, Dw, Fw, wfs, bf.reshape(1, OUT))
    return out


# speedup vs baseline: 8.5988x; 1.0154x over previous
import jax
import jax.numpy as jnp
from jax import lax
from jax.experimental import pallas as pl
from jax.experimental.pallas import tpu as pltpu
from jax.experimental.pallas import tpu_sc as plsc

N = 10000
D = 128
E = 320000
P = 16
OUT = 128

NW = 32
EPW = E // NW
CHUNK = 400
NB_PREP = 10
GB = 80
BE = 2000


def _prep_body(x_ref, wcat_ref, bcat_ref, wh1_ref, wh2_ref,
               t1_ref, t2_ref):
    xb = x_ref[...]
    z = jnp.maximum(
        jnp.dot(xb, wcat_ref[...], preferred_element_type=jnp.float32)
        + bcat_ref[...], 0.0)
    h1 = jnp.dot(z, wh1_ref[...], preferred_element_type=jnp.float32)
    h2 = jnp.dot(z, wh2_ref[...], preferred_element_type=jnp.float32)
    z4 = z[:, 48:64]
    rb = z.shape[0]

    def pk(a):
        half = a.shape[1] // 2
        lo = lax.bitcast_convert_type(
            a[:, :half].astype(jnp.bfloat16), jnp.uint16).astype(jnp.uint32)
        hi = lax.bitcast_convert_type(
            a[:, half:].astype(jnp.bfloat16), jnp.uint16).astype(jnp.uint32)
        return lax.bitcast_convert_type(lo | (hi << 16), jnp.int32)

    n2 = jnp.sum(xb * xb, axis=1, keepdims=True)
    xh = xb / jnp.maximum(jnp.sqrt(n2), 1e-8)
    xq = lax.bitcast_convert_type(
        xh.astype(jnp.float8_e4m3fn), jnp.uint8).astype(jnp.uint32)
    xw = lax.bitcast_convert_type(
        xq[:, 0:32] | (xq[:, 32:64] << 8) | (xq[:, 64:96] << 16)
        | (xq[:, 96:128] << 24), jnp.int32)
    pad = jnp.zeros((rb, 24), jnp.int32)
    t1_ref[...] = jnp.concatenate([pk(h1), pk(z4), xw, pad], axis=1)
    t2_ref[...] = jnp.concatenate([pk(h2), pk(z4), xw, pad], axis=1)


def _prep_call(x, wcat, bcat, wh1, wh2):
    rb = N // NB_PREP
    return pl.pallas_call(
        _prep_body,
        grid=(NB_PREP,),
        in_specs=[
            pl.BlockSpec((rb, D), lambda i: (i, 0)),
            pl.BlockSpec((D, 64), lambda i: (0, 0)),
            pl.BlockSpec((1, 64), lambda i: (0, 0)),
            pl.BlockSpec((64, OUT), lambda i: (0, 0)),
            pl.BlockSpec((64, OUT), lambda i: (0, 0)),
        ],
        out_specs=[
            pl.BlockSpec((rb, D), lambda i: (i, 0)),
            pl.BlockSpec((rb, D), lambda i: (i, 0)),
        ],
        out_shape=[
            jax.ShapeDtypeStruct((N, D), jnp.int32),
            jax.ShapeDtypeStruct((N, D), jnp.int32),
        ],
    )(x, wcat, bcat, wh1, wh2)


def _sc_body(t1, t2, rowv, colv,
             ga, gb,
             idxr, idxc, bufa, bufb, sem):
    cid = lax.axis_index("c")
    sid = lax.axis_index("s")
    wid = sid * 2 + cid
    base0 = wid * EPW

    def chunk_body(ci, carry):
        base = base0 + ci * CHUNK
        pltpu.sync_copy(rowv.at[pl.ds(base, CHUNK)], idxr)
        pltpu.sync_copy(colv.at[pl.ds(base, CHUNK)], idxc)
        d1 = pltpu.async_copy(t1.at[idxr], bufa, sem)
        d2 = pltpu.async_copy(t2.at[idxc], bufb, sem)
        d1.wait()
        d2.wait()
        pltpu.sync_copy(bufa, ga.at[pl.ds(base, CHUNK)])
        pltpu.sync_copy(bufb, gb.at[pl.ds(base, CHUNK)])
        return carry

    lax.fori_loop(0, EPW // CHUNK, chunk_body, 0)


def _sc_call(t1, t2, rowv, colv):
    mesh = plsc.VectorSubcoreMesh(core_axis_name="c", subcore_axis_name="s")
    fn = pl.kernel(
        _sc_body,
        out_type=[
            jax.ShapeDtypeStruct((E, D), jnp.int32),
            jax.ShapeDtypeStruct((E, D), jnp.int32),
        ],
        mesh=mesh,
        scratch_types=[
            pltpu.VMEM((CHUNK,), jnp.int32),
            pltpu.VMEM((CHUNK,), jnp.int32),
            pltpu.VMEM((CHUNK, D), jnp.int32),
            pltpu.VMEM((CHUNK, D), jnp.int32),
            pltpu.SemaphoreType.DMA,
        ],
    )
    return fn(t1, t2, rowv, colv)


def _lo(w):
    return lax.bitcast_convert_type(lax.shift_left(w, 16), jnp.float32)


def _hi(w):
    return lax.bitcast_convert_type(w & jnp.int32(-65536), jnp.float32)


def _f8(w, k):
    b = lax.shift_right_logical(w, 8 * k) & jnp.int32(0xFF)
    return lax.bitcast_convert_type(
        b.astype(jnp.uint8), jnp.float8_e4m3fn).astype(jnp.float32)


def _final_body(ga_ref, gb_ref, ea_ref,
                dw_ref, fw_ref, wfs_ref, bf_ref, out_ref):
    ga = ga_ref[...]
    gb = gb_ref[...]
    dw = dw_ref[...]
    fw = fw_ref[...]
    wfs = wfs_ref[...]
    bf = bf_ref[...]
    ea = ea_ref[...]
    gaz = ga[:, 64:72]
    gbz = gb[:, 64:72]
    q_lo = jnp.sqrt(_lo(gaz) * _lo(gbz))
    q_hi = jnp.sqrt(_hi(gaz) * _hi(gbz))
    gax = ga[:, 72:104]
    gbx = gb[:, 72:104]
    prods = [_f8(gax, k) * _f8(gbx, k) for k in range(4)]
    psum = (prods[0] + prods[1]) + (prods[2] + prods[3])
    for half, sl in ((0, slice(0, 64)), (1, slice(64, 128))):
        ext = _lo if half == 0 else _hi
        acc = ext(ga[:, 0:64]) + ext(gb[:, 0:64]) + bf[:, sl]
        acc = acc + jnp.dot(q_lo, dw[0:8, sl],
                            preferred_element_type=jnp.float32)
        acc = acc + jnp.dot(q_hi, dw[8:16, sl],
                            preferred_element_type=jnp.float32)
        acc = acc + jnp.dot(ea, fw[:, sl],
                            preferred_element_type=jnp.float32)
        wcos = jnp.broadcast_to(wfs[:, sl], (32, 64))
        acc = acc + jnp.dot(psum, wcos, preferred_element_type=jnp.float32)
        out_ref[:, sl] = jnp.maximum(acc, 0.0)


def _final_call(ga, gb, ea, dw, fw, wfs, bfv):
    nb = E // BE
    return pl.pallas_call(
        _final_body,
        grid=(nb,),
        in_specs=[
            pl.BlockSpec((BE, D), lambda i: (i, 0)),
            pl.BlockSpec((BE, D), lambda i: (i, 0)),
            pl.BlockSpec((BE, P), lambda i: (i, 0)),
            pl.BlockSpec((P, OUT), lambda i: (0, 0)),
            pl.BlockSpec((P, OUT), lambda i: (0, 0)),
            pl.BlockSpec((1, OUT), lambda i: (0, 0)),
            pl.BlockSpec((1, OUT), lambda i: (0, 0)),
        ],
        out_specs=pl.BlockSpec((BE, OUT), lambda i: (i, 0)),
        out_shape=jax.ShapeDtypeStruct((E, OUT), jnp.float32),
    )(ga, gb, ea, dw, fw, wfs, bfv)


def kernel(x, edge_index, edge_attr, W1, b1, W2, b2, W3, b3, W4, b4, Wf, bf):
    row = edge_index[0].astype(jnp.int32)
    col = edge_index[1].astype(jnp.int32)
    wcat = jnp.concatenate([W1, W2, W3, W4], axis=1)
    bcat = jnp.concatenate([b1, b2, b3, b4]).reshape(1, 4 * P)
    A = Wf[0:P]
    Bw = Wf[P:2 * P]
    Cw = Wf[2 * P:3 * P]
    Dw = Wf[3 * P:4 * P]
    wfs = Wf[4 * P:4 * P + 1]
    Fw = Wf[4 * P + 1:]
    wh1 = jnp.zeros((4 * P, OUT), jnp.float32).at[0:P].set(A).at[2 * P:3 * P].set(Cw)
    wh2 = jnp.zeros((4 * P, OUT), jnp.float32).at[P:2 * P].set(Bw).at[2 * P:3 * P].set(-Cw)
    t1, t2 = _prep_call(x, wcat, bcat, wh1, wh2)
    ga, gb = _sc_call(t1, t2, row, col)
    out = _final_call(ga, gb, edge_attr, Dw, Fw, wfs, bf.reshape(1, OUT))
    return out


# speedup vs baseline: 10.0934x; 1.1738x over previous
import jax
import jax.numpy as jnp
from jax import lax
from jax.experimental import pallas as pl
from jax.experimental.pallas import tpu as pltpu
from jax.experimental.pallas import tpu_sc as plsc

N = 10000
D = 128
E = 320000
P = 16
OUT = 128

NW = 32
NH = 2
E2 = E // NH
EPW = E2 // NW
CHUNK = 200
NB_PREP = 10
BE = 3200


def _prep_body(x_ref, wcat_ref, bcat_ref, wh1_ref, wh2_ref,
               t1_ref, t2_ref):
    xb = x_ref[...]
    z = jnp.maximum(
        jnp.dot(xb, wcat_ref[...], preferred_element_type=jnp.float32)
        + bcat_ref[...], 0.0)
    h1 = jnp.dot(z, wh1_ref[...], preferred_element_type=jnp.float32)
    h2 = jnp.dot(z, wh2_ref[...], preferred_element_type=jnp.float32)
    z4 = z[:, 48:64]
    rb = z.shape[0]

    def pk(a):
        half = a.shape[1] // 2
        lo = lax.bitcast_convert_type(
            a[:, :half].astype(jnp.bfloat16), jnp.uint16).astype(jnp.uint32)
        hi = lax.bitcast_convert_type(
            a[:, half:].astype(jnp.bfloat16), jnp.uint16).astype(jnp.uint32)
        return lax.bitcast_convert_type(lo | (hi << 16), jnp.int32)

    n2 = jnp.sum(xb * xb, axis=1, keepdims=True)
    xh = xb / jnp.maximum(jnp.sqrt(n2), 1e-8)
    xq = lax.bitcast_convert_type(
        xh.astype(jnp.float8_e4m3fn), jnp.uint8).astype(jnp.uint32)
    xw = lax.bitcast_convert_type(
        xq[:, 0:32] | (xq[:, 32:64] << 8) | (xq[:, 64:96] << 16)
        | (xq[:, 96:128] << 24), jnp.int32)
    pad = jnp.zeros((rb, 24), jnp.int32)
    t1_ref[...] = jnp.concatenate([pk(h1), pk(z4), xw, pad], axis=1)
    t2_ref[...] = jnp.concatenate([pk(h2), pk(z4), xw, pad], axis=1)


def _prep_call(x, wcat, bcat, wh1, wh2):
    rb = N // NB_PREP
    return pl.pallas_call(
        _prep_body,
        grid=(NB_PREP,),
        in_specs=[
            pl.BlockSpec((rb, D), lambda i: (i, 0)),
            pl.BlockSpec((D, 64), lambda i: (0, 0)),
            pl.BlockSpec((1, 64), lambda i: (0, 0)),
            pl.BlockSpec((64, OUT), lambda i: (0, 0)),
            pl.BlockSpec((64, OUT), lambda i: (0, 0)),
        ],
        out_specs=[
            pl.BlockSpec((rb, D), lambda i: (i, 0)),
            pl.BlockSpec((rb, D), lambda i: (i, 0)),
        ],
        out_shape=[
            jax.ShapeDtypeStruct((N, D), jnp.int32),
            jax.ShapeDtypeStruct((N, D), jnp.int32),
        ],
    )(x, wcat, bcat, wh1, wh2)


def _sc_body(t1, t2, rowv, colv,
             ga, gb,
             idxr, idxc, bufa, bufb, sem):
    cid = lax.axis_index("c")
    sid = lax.axis_index("s")
    wid = sid * 2 + cid
    base0 = wid * EPW

    def chunk_body(ci, carry):
        base = base0 + ci * CHUNK
        pltpu.sync_copy(rowv.at[pl.ds(base, CHUNK)], idxr)
        pltpu.sync_copy(colv.at[pl.ds(base, CHUNK)], idxc)
        d1 = pltpu.async_copy(t1.at[idxr], bufa, sem)
        d2 = pltpu.async_copy(t2.at[idxc], bufb, sem)
        d1.wait()
        d2.wait()
        pltpu.sync_copy(bufa, ga.at[pl.ds(base, CHUNK)])
        pltpu.sync_copy(bufb, gb.at[pl.ds(base, CHUNK)])
        return carry

    lax.fori_loop(0, EPW // CHUNK, chunk_body, 0)


def _sc_call(t1, t2, rowv, colv):
    mesh = plsc.VectorSubcoreMesh(core_axis_name="c", subcore_axis_name="s")
    fn = pl.kernel(
        _sc_body,
        out_type=[
            jax.ShapeDtypeStruct((E2, D), jnp.int32),
            jax.ShapeDtypeStruct((E2, D), jnp.int32),
        ],
        mesh=mesh,
        scratch_types=[
            pltpu.VMEM((CHUNK,), jnp.int32),
            pltpu.VMEM((CHUNK,), jnp.int32),
            pltpu.VMEM((CHUNK, D), jnp.int32),
            pltpu.VMEM((CHUNK, D), jnp.int32),
            pltpu.SemaphoreType.DMA,
        ],
    )
    return fn(t1, t2, rowv, colv)


def _lo(w):
    return lax.bitcast_convert_type(lax.shift_left(w, 16), jnp.float32)


def _hi(w):
    return lax.bitcast_convert_type(w & jnp.int32(-65536), jnp.float32)


def _f8(w, k):
    b = lax.shift_right_logical(w, 8 * k) & jnp.int32(0xFF)
    return lax.bitcast_convert_type(
        b.astype(jnp.uint8), jnp.float8_e4m3fn).astype(jnp.float32)


def _final_body(prev_ref, ga_ref, gb_ref, eat_ref,
                dw_ref, fw_ref, wfs_ref, bf_ref, out_ref):
    del prev_ref
    ga = ga_ref[...]
    gb = gb_ref[...]
    dw = dw_ref[...]
    fw = fw_ref[...]
    wfs = wfs_ref[...]
    bf = bf_ref[...]
    eat = eat_ref[...]
    gaz = ga[:, 64:72]
    gbz = gb[:, 64:72]
    q_lo = jnp.sqrt(_lo(gaz) * _lo(gbz))
    q_hi = jnp.sqrt(_hi(gaz) * _hi(gbz))
    gax = ga[:, 72:104]
    gbx = gb[:, 72:104]
    prods = [_f8(gax, k) * _f8(gbx, k) for k in range(4)]
    psum = (prods[0] + prods[1]) + (prods[2] + prods[3])
    for half, sl in ((0, slice(0, 64)), (1, slice(64, 128))):
        ext = _lo if half == 0 else _hi
        acc = ext(ga[:, 0:64]) + ext(gb[:, 0:64]) + bf[:, sl]
        acc = acc + jnp.dot(q_lo, dw[0:8, sl],
                            preferred_element_type=jnp.float32)
        acc = acc + jnp.dot(q_hi, dw[8:16, sl],
                            preferred_element_type=jnp.float32)
        acc = acc + lax.dot_general(eat, fw[:, sl], (((0,), (0,)), ((), ())),
                                    preferred_element_type=jnp.float32)
        wcos = jnp.broadcast_to(wfs[:, sl], (32, 64))
        acc = acc + jnp.dot(psum, wcos, preferred_element_type=jnp.float32)
        out_ref[:, sl] = jnp.maximum(acc, 0.0)


def _final_call(prev, ga, gb, eat, dw, fw, wfs, bfv, h):
    nbh = E2 // BE
    body = _final_body if prev is not None else (
        lambda *refs: _final_body(None, *refs))
    in_specs = [
        pl.BlockSpec((BE, D), lambda i: (i, 0)),
        pl.BlockSpec((BE, D), lambda i: (i, 0)),
        pl.BlockSpec((P, BE), lambda i, _h=h: (0, i + _h * (E2 // BE))),
        pl.BlockSpec((P, OUT), lambda i: (0, 0)),
        pl.BlockSpec((P, OUT), lambda i: (0, 0)),
        pl.BlockSpec((1, OUT), lambda i: (0, 0)),
        pl.BlockSpec((1, OUT), lambda i: (0, 0)),
    ]
    args = (ga, gb, eat, dw, fw, wfs, bfv)
    aliases = {}
    if prev is not None:
        in_specs = [pl.BlockSpec((8, OUT), lambda i: (0, 0))] + in_specs
        args = (prev,) + args
        aliases = {0: 0}
    return pl.pallas_call(
        body,
        grid=(nbh,),
        in_specs=in_specs,
        out_specs=pl.BlockSpec((BE, OUT),
                               lambda i, _h=h: (i + _h * (E2 // BE), 0)),
        out_shape=jax.ShapeDtypeStruct((E, OUT), jnp.float32),
        input_output_aliases=aliases,
    )(*args)


def kernel(x, edge_index, edge_attr, W1, b1, W2, b2, W3, b3, W4, b4, Wf, bf):
    row = edge_index[0].astype(jnp.int32)
    col = edge_index[1].astype(jnp.int32)
    eat = edge_attr.T
    wcat = jnp.concatenate([W1, W2, W3, W4], axis=1)
    bcat = jnp.concatenate([b1, b2, b3, b4]).reshape(1, 4 * P)
    A = Wf[0:P]
    Bw = Wf[P:2 * P]
    Cw = Wf[2 * P:3 * P]
    Dw = Wf[3 * P:4 * P]
    wfs = Wf[4 * P:4 * P + 1]
    Fw = Wf[4 * P + 1:]
    wh1 = jnp.zeros((4 * P, OUT), jnp.float32).at[0:P].set(A).at[2 * P:3 * P].set(Cw)
    wh2 = jnp.zeros((4 * P, OUT), jnp.float32).at[P:2 * P].set(Bw).at[2 * P:3 * P].set(-Cw)
    t1, t2 = _prep_call(x, wcat, bcat, wh1, wh2)
    bfv = bf.reshape(1, OUT)
    parts = []
    for h in range(NH):
        sl = slice(h * E2, (h + 1) * E2)
        parts.append(_sc_call(t1, t2, row[sl], col[sl]))
    out = None
    for h in range(NH):
        ga, gb = parts[h]
        out = _final_call(out, ga, gb, eat, Dw, Fw, wfs, bfv, h)
    return out


# speedup vs baseline: 11.8102x; 1.1701x over previous
import jax
import jax.numpy as jnp
from jax import lax
from jax.experimental import pallas as pl
from jax.experimental.pallas import tpu as pltpu
from jax.experimental.pallas import tpu_sc as plsc

N = 10000
D = 128
E = 320000
P = 16
OUT = 128

NW = 32
NH = 5
E2 = E // NH
EPW = E2 // NW
CHUNK = 200
NB_PREP = 10
BE = 2560


def _prep_body(x_ref, wcat_ref, bcat_ref, wh1_ref, wh2_ref,
               t1_ref, t2_ref):
    xb = x_ref[...]
    z = jnp.maximum(
        jnp.dot(xb, wcat_ref[...], preferred_element_type=jnp.float32)
        + bcat_ref[...], 0.0)
    h1 = jnp.dot(z, wh1_ref[...], preferred_element_type=jnp.float32)
    h2 = jnp.dot(z, wh2_ref[...], preferred_element_type=jnp.float32)
    z4 = jnp.sqrt(z[:, 48:64])
    rb = z.shape[0]

    def pk(a):
        half = a.shape[1] // 2
        lo = lax.bitcast_convert_type(
            a[:, :half].astype(jnp.bfloat16), jnp.uint16).astype(jnp.uint32)
        hi = lax.bitcast_convert_type(
            a[:, half:].astype(jnp.bfloat16), jnp.uint16).astype(jnp.uint32)
        return lax.bitcast_convert_type(lo | (hi << 16), jnp.int32)

    n2 = jnp.sum(xb * xb, axis=1, keepdims=True)
    xh = xb / jnp.maximum(jnp.sqrt(n2), 1e-8)
    xq = lax.bitcast_convert_type(
        xh.astype(jnp.float8_e4m3fn), jnp.uint8).astype(jnp.uint32)
    xw = lax.bitcast_convert_type(
        xq[:, 0:32] | (xq[:, 32:64] << 8) | (xq[:, 64:96] << 16)
        | (xq[:, 96:128] << 24), jnp.int32)
    pad = jnp.zeros((rb, 24), jnp.int32)
    t1_ref[...] = jnp.concatenate([pk(h1), pk(z4), xw, pad], axis=1)
    t2_ref[...] = jnp.concatenate([pk(h2), pk(z4), xw, pad], axis=1)


def _prep_call(x, wcat, bcat, wh1, wh2):
    rb = N // NB_PREP
    return pl.pallas_call(
        _prep_body,
        grid=(NB_PREP,),
        in_specs=[
            pl.BlockSpec((rb, D), lambda i: (i, 0)),
            pl.BlockSpec((D, 64), lambda i: (0, 0)),
            pl.BlockSpec((1, 64), lambda i: (0, 0)),
            pl.BlockSpec((64, OUT), lambda i: (0, 0)),
            pl.BlockSpec((64, OUT), lambda i: (0, 0)),
        ],
        out_specs=[
            pl.BlockSpec((rb, D), lambda i: (i, 0)),
            pl.BlockSpec((rb, D), lambda i: (i, 0)),
        ],
        out_shape=[
            jax.ShapeDtypeStruct((N, D), jnp.int32),
            jax.ShapeDtypeStruct((N, D), jnp.int32),
        ],
    )(x, wcat, bcat, wh1, wh2)


def _sc_body(t1, t2, rowv, colv,
             ga, gb,
             idxr, idxc, bufa, bufb, sem):
    cid = lax.axis_index("c")
    sid = lax.axis_index("s")
    wid = sid * 2 + cid
    base0 = wid * EPW

    def chunk_body(ci, carry):
        base = base0 + ci * CHUNK
        pltpu.sync_copy(rowv.at[pl.ds(base, CHUNK)], idxr)
        pltpu.sync_copy(colv.at[pl.ds(base, CHUNK)], idxc)
        d1 = pltpu.async_copy(t1.at[idxr], bufa, sem)
        d2 = pltpu.async_copy(t2.at[idxc], bufb, sem)
        d1.wait()
        d2.wait()
        pltpu.sync_copy(bufa, ga.at[pl.ds(base, CHUNK)])
        pltpu.sync_copy(bufb, gb.at[pl.ds(base, CHUNK)])
        return carry

    lax.fori_loop(0, EPW // CHUNK, chunk_body, 0)


def _sc_call(t1, t2, rowv, colv):
    mesh = plsc.VectorSubcoreMesh(core_axis_name="c", subcore_axis_name="s")
    fn = pl.kernel(
        _sc_body,
        out_type=[
            jax.ShapeDtypeStruct((E2, D), jnp.int32),
            jax.ShapeDtypeStruct((E2, D), jnp.int32),
        ],
        mesh=mesh,
        scratch_types=[
            pltpu.VMEM((CHUNK,), jnp.int32),
            pltpu.VMEM((CHUNK,), jnp.int32),
            pltpu.VMEM((CHUNK, D), jnp.int32),
            pltpu.VMEM((CHUNK, D), jnp.int32),
            pltpu.SemaphoreType.DMA,
        ],
    )
    return fn(t1, t2, rowv, colv)


def _lo(w):
    return lax.bitcast_convert_type(lax.shift_left(w, 16), jnp.float32)


def _hi(w):
    return lax.bitcast_convert_type(w & jnp.int32(-65536), jnp.float32)


def _f8(w, k):
    b = lax.shift_right_logical(w, 8 * k) & jnp.int32(0xFF)
    return lax.bitcast_convert_type(
        b.astype(jnp.uint8), jnp.float8_e4m3fn).astype(jnp.float32)


def _final_body(prev_ref, ga_ref, gb_ref, eat_ref,
                dw_ref, fw_ref, wfs_ref, bf_ref, out_ref):
    del prev_ref
    ga = ga_ref[...]
    gb = gb_ref[...]
    dw = dw_ref[...]
    fw = fw_ref[...]
    wfs = wfs_ref[...]
    bf = bf_ref[...]
    eat = eat_ref[...]
    gaz = ga[:, 64:72]
    gbz = gb[:, 64:72]
    q_lo = _lo(gaz) * _lo(gbz)
    q_hi = _hi(gaz) * _hi(gbz)
    gax = ga[:, 72:104]
    gbx = gb[:, 72:104]
    prods = [_f8(gax, k) * _f8(gbx, k) for k in range(4)]
    psum = (prods[0] + prods[1]) + (prods[2] + prods[3])
    for half, sl in ((0, slice(0, 64)), (1, slice(64, 128))):
        ext = _lo if half == 0 else _hi
        acc = ext(ga[:, 0:64]) + ext(gb[:, 0:64]) + bf[:, sl]
        acc = acc + jnp.dot(q_lo, dw[0:8, sl],
                            preferred_element_type=jnp.float32)
        acc = acc + jnp.dot(q_hi, dw[8:16, sl],
                            preferred_element_type=jnp.float32)
        acc = acc + lax.dot_general(eat, fw[:, sl], (((0,), (0,)), ((), ())),
                                    preferred_element_type=jnp.float32)
        wcos = jnp.broadcast_to(wfs[:, sl], (32, 64))
        acc = acc + jnp.dot(psum, wcos, preferred_element_type=jnp.float32)
        out_ref[:, sl] = jnp.maximum(acc, 0.0)


def _final_call(prev, ga, gb, eat, dw, fw, wfs, bfv, h):
    nbh = E2 // BE
    body = _final_body if prev is not None else (
        lambda *refs: _final_body(None, *refs))
    in_specs = [
        pl.BlockSpec((BE, D), lambda i: (i, 0)),
        pl.BlockSpec((BE, D), lambda i: (i, 0)),
        pl.BlockSpec((P, BE), lambda i, _h=h: (0, i + _h * (E2 // BE))),
        pl.BlockSpec((P, OUT), lambda i: (0, 0)),
        pl.BlockSpec((P, OUT), lambda i: (0, 0)),
        pl.BlockSpec((1, OUT), lambda i: (0, 0)),
        pl.BlockSpec((1, OUT), lambda i: (0, 0)),
    ]
    args = (ga, gb, eat, dw, fw, wfs, bfv)
    aliases = {}
    if prev is not None:
        in_specs = [pl.BlockSpec((8, OUT), lambda i: (0, 0))] + in_specs
        args = (prev,) + args
        aliases = {0: 0}
    return pl.pallas_call(
        body,
        grid=(nbh,),
        in_specs=in_specs,
        out_specs=pl.BlockSpec((BE, OUT),
                               lambda i, _h=h: (i + _h * (E2 // BE), 0)),
        out_shape=jax.ShapeDtypeStruct((E, OUT), jnp.float32),
        input_output_aliases=aliases,
    )(*args)


def kernel(x, edge_index, edge_attr, W1, b1, W2, b2, W3, b3, W4, b4, Wf, bf):
    row = edge_index[0].astype(jnp.int32)
    col = edge_index[1].astype(jnp.int32)
    eat = edge_attr.T
    wcat = jnp.concatenate([W1, W2, W3, W4], axis=1)
    bcat = jnp.concatenate([b1, b2, b3, b4]).reshape(1, 4 * P)
    A = Wf[0:P]
    Bw = Wf[P:2 * P]
    Cw = Wf[2 * P:3 * P]
    Dw = Wf[3 * P:4 * P]
    wfs = Wf[4 * P:4 * P + 1]
    Fw = Wf[4 * P + 1:]
    wh1 = jnp.zeros((4 * P, OUT), jnp.float32).at[0:P].set(A).at[2 * P:3 * P].set(Cw)
    wh2 = jnp.zeros((4 * P, OUT), jnp.float32).at[P:2 * P].set(Bw).at[2 * P:3 * P].set(-Cw)
    t1, t2 = _prep_call(x, wcat, bcat, wh1, wh2)
    bfv = bf.reshape(1, OUT)
    parts = []
    for h in range(NH):
        sl = slice(h * E2, (h + 1) * E2)
        parts.append(_sc_call(t1, t2, row[sl], col[sl]))
    out = None
    for h in range(NH):
        ga, gb = parts[h]
        out = _final_call(out, ga, gb, eat, Dw, Fw, wfs, bfv, h)
    return out
